# baseline (speedup 1.0000x reference)
import functools

import jax
import jax.numpy as jnp
from jax import lax
from jax.experimental import pallas as pl
from jax.experimental.pallas import tpu as pltpu
from jax.experimental.pallas import tpu_sc as plsc

BATCH = 16384
EMBED_DIM = 64
CLS_DIM = 32
NUM_ACT = 1000000
NC = 2
NS = 16
NW = NC * NS
B_PER_W = BATCH // NW
CHUNK = 128
N_CHUNK = B_PER_W // CHUNK

TL = 4096
TGRID = (NUM_ACT + TL - 1) // TL


def _tr_body(in_ref, o_ref):
    x = in_ref[...]
    o_ref[:, :EMBED_DIM] = lax.dot_general(
        x, jnp.eye(EMBED_DIM, dtype=jnp.float32),
        dimension_numbers=(((0,), (0,)), ((), ())),
        precision=lax.Precision.HIGHEST,
        preferred_element_type=jnp.float32)


def _tc_pairize(embT):
    return pl.pallas_call(
        _tr_body,
        grid=(TGRID,),
        in_specs=[pl.BlockSpec((EMBED_DIM, TL), lambda i: (0, i))],
        out_specs=pl.BlockSpec((TL, 128), lambda i: (i, 0)),
        out_shape=jax.ShapeDtypeStruct((NUM_ACT, 128), jnp.float32),
    )(embT)


@functools.partial(
    pl.kernel,
    out_type=(
        jax.ShapeDtypeStruct((BATCH, 128), jnp.float32),
        jax.ShapeDtypeStruct((BATCH, 128), jnp.float32),
    ),
    mesh=plsc.VectorSubcoreMesh(core_axis_name="c", subcore_axis_name="s"),
    compiler_params=pltpu.CompilerParams(use_tc_tiling_on_sc=True),
    scratch_types=[
        pltpu.VMEM((B_PER_W,), jnp.int32),
        pltpu.VMEM((B_PER_W,), jnp.int32),
        pltpu.VMEM((B_PER_W, 128), jnp.float32),
        pltpu.VMEM((B_PER_W // 2, 128), jnp.float32),
        pltpu.SemaphoreType.DMA,
        pltpu.SemaphoreType.DMA,
    ],
)
def _sc_gather(ids_hbm, cls_hbm, emb_hbm, clsemb_hbm, act_out, cls_out,
               ids_v, clsids_v, act_rows, cls_rows, sem_a, sem_c):
    wid = lax.axis_index("s") * NC + lax.axis_index("c")
    base = wid * B_PER_W
    pltpu.sync_copy(ids_hbm.at[pl.ds(base, B_PER_W)], ids_v)
    pltpu.sync_copy(cls_hbm.at[pl.ds(base, B_PER_W)], clsids_v)
    act_copies = []
    for j in range(N_CHUNK):
        act_copies.append(pltpu.async_copy(
            emb_hbm.at[ids_v.at[pl.ds(j * CHUNK, CHUNK)]],
            act_rows.at[pl.ds(j * CHUNK, CHUNK)], sem_a))
    for r in range(2):
        cls_copies = []
        for j in range(2):
            cls_copies.append(pltpu.async_copy(
                clsemb_hbm.at[clsids_v.at[pl.ds((2 * r + j) * CHUNK, CHUNK)]],
                cls_rows.at[pl.ds(j * CHUNK, CHUNK)], sem_c))
        for c in cls_copies:
            c.wait()
        pltpu.sync_copy(cls_rows,
                        cls_out.at[pl.ds(base + r * (B_PER_W // 2),
                                         B_PER_W // 2)])
    for c in act_copies:
        c.wait()
    pltpu.sync_copy(act_rows, act_out.at[pl.ds(base, B_PER_W)])


def _mm_body(act2_ref, cls4_ref, clsm_ref, w1_ref, w2_ref, b_ref,
             o_ref):
    clsm = clsm_ref[...]
    act = act2_ref[:, :EMBED_DIM]
    c = cls4_ref[...]
    cls_sel = c[:, 0:CLS_DIM] * (clsm == 0.0)
    cls_sel += c[:, CLS_DIM:2 * CLS_DIM] * (clsm == 1.0)
    cls_sel += c[:, 2 * CLS_DIM:3 * CLS_DIM] * (clsm == 2.0)
    cls_sel += c[:, 3 * CLS_DIM:] * (clsm == 3.0)
    acc = jnp.dot(act, w1_ref[...],
                  preferred_element_type=jnp.float32,
                  precision=lax.Precision.HIGHEST)
    acc += jnp.dot(cls_sel, w2_ref[...],
                   preferred_element_type=jnp.float32,
                   precision=lax.Precision.HIGHEST)
    o_ref[...] = acc + b_ref[...]


def _tc_project(act2, cls4, clsm, w1, w2, b2d):
    blk = 2048
    grid = (BATCH // blk,)
    return pl.pallas_call(
        _mm_body,
        grid=grid,
        in_specs=[
            pl.BlockSpec((blk, 128), lambda i: (i, 0)),
            pl.BlockSpec((blk, 128), lambda i: (i, 0)),
            pl.BlockSpec((blk, 1), lambda i: (i, 0)),
            pl.BlockSpec((EMBED_DIM, EMBED_DIM), lambda i: (0, 0)),
            pl.BlockSpec((CLS_DIM, EMBED_DIM), lambda i: (0, 0)),
            pl.BlockSpec((1, EMBED_DIM), lambda i: (0, 0)),
        ],
        out_specs=pl.BlockSpec((blk, EMBED_DIM), lambda i: (i, 0)),
        out_shape=jax.ShapeDtypeStruct((BATCH, EMBED_DIM), jnp.float32),
    )(act2, cls4, clsm, w1, w2, b2d)


def kernel(activity_ids, activity_classes, embedding, class_embedding, W, b):
    ids = activity_ids.astype(jnp.int32)
    cls = activity_classes.astype(jnp.int32)
    emb2 = _tc_pairize(embedding.T)
    cls2 = class_embedding.reshape(250, 128)
    act2, cls4 = _sc_gather(ids, cls // 4, emb2, cls2)
    clsm = (cls % 4).astype(jnp.float32).reshape(BATCH, 1)
    return _tc_project(act2, cls4, clsm,
                       W[:EMBED_DIM], W[EMBED_DIM:], b.reshape(1, EMBED_DIM))


# speedup vs baseline: 1.3807x; 1.3807x over previous
import functools

import jax
import jax.numpy as jnp
from jax import lax
from jax.experimental import pallas as pl
from jax.experimental.pallas import tpu as pltpu
from jax.experimental.pallas import tpu_sc as plsc

BATCH = 16384
EMBED_DIM = 64
CLS_DIM = 32
NUM_ACT = 1000000
NC = 2
NS = 16
NW = NC * NS
B_PER_W = BATCH // NW
CHUNK = 128
N_CHUNK = B_PER_W // CHUNK

TL = 4096
TGRID = (NUM_ACT + TL - 1) // TL


def _tr_body(in_ref, o_ref):
    x = in_ref[...].astype(jnp.bfloat16)
    o_ref[:, :EMBED_DIM] = lax.dot_general(
        x, jnp.eye(EMBED_DIM, dtype=jnp.bfloat16),
        dimension_numbers=(((0,), (0,)), ((), ())),
        preferred_element_type=jnp.float32)


def _tc_pairize(embT):
    return pl.pallas_call(
        _tr_body,
        grid=(TGRID,),
        in_specs=[pl.BlockSpec((EMBED_DIM, TL), lambda i: (0, i))],
        out_specs=pl.BlockSpec((TL, 128), lambda i: (i, 0)),
        out_shape=jax.ShapeDtypeStruct((NUM_ACT, 128), jnp.float32),
    )(embT)


@functools.partial(
    pl.kernel,
    out_type=(
        jax.ShapeDtypeStruct((BATCH, 128), jnp.float32),
        jax.ShapeDtypeStruct((BATCH, 128), jnp.float32),
    ),
    mesh=plsc.VectorSubcoreMesh(core_axis_name="c", subcore_axis_name="s"),
    compiler_params=pltpu.CompilerParams(use_tc_tiling_on_sc=True),
    scratch_types=[
        pltpu.VMEM((B_PER_W,), jnp.int32),
        pltpu.VMEM((B_PER_W,), jnp.int32),
        pltpu.VMEM((B_PER_W, 128), jnp.float32),
        pltpu.VMEM((B_PER_W // 2, 128), jnp.float32),
        pltpu.SemaphoreType.DMA,
        pltpu.SemaphoreType.DMA,
    ],
)
def _sc_gather(ids_hbm, cls_hbm, emb_hbm, clsemb_hbm, act_out, cls_out,
               ids_v, clsids_v, act_rows, cls_rows, sem_a, sem_c):
    wid = lax.axis_index("s") * NC + lax.axis_index("c")
    base = wid * B_PER_W
    pltpu.sync_copy(ids_hbm.at[pl.ds(base, B_PER_W)], ids_v)
    pltpu.sync_copy(cls_hbm.at[pl.ds(base, B_PER_W)], clsids_v)
    act_copies = []
    for j in range(N_CHUNK):
        act_copies.append(pltpu.async_copy(
            emb_hbm.at[ids_v.at[pl.ds(j * CHUNK, CHUNK)]],
            act_rows.at[pl.ds(j * CHUNK, CHUNK)], sem_a))
    for r in range(2):
        cls_copies = []
        for j in range(2):
            cls_copies.append(pltpu.async_copy(
                clsemb_hbm.at[clsids_v.at[pl.ds((2 * r + j) * CHUNK, CHUNK)]],
                cls_rows.at[pl.ds(j * CHUNK, CHUNK)], sem_c))
        for c in cls_copies:
            c.wait()
        pltpu.sync_copy(cls_rows,
                        cls_out.at[pl.ds(base + r * (B_PER_W // 2),
                                         B_PER_W // 2)])
    for c in act_copies:
        c.wait()
    pltpu.sync_copy(act_rows, act_out.at[pl.ds(base, B_PER_W)])


def _mm_body(act2_ref, cls4_ref, clsm_ref, w1_ref, w2_ref, b_ref,
             o_ref):
    clsm = clsm_ref[...]
    act = act2_ref[:, :EMBED_DIM]
    c = cls4_ref[...]
    cls_sel = c[:, 0:CLS_DIM] * (clsm == 0.0)
    cls_sel += c[:, CLS_DIM:2 * CLS_DIM] * (clsm == 1.0)
    cls_sel += c[:, 2 * CLS_DIM:3 * CLS_DIM] * (clsm == 2.0)
    cls_sel += c[:, 3 * CLS_DIM:] * (clsm == 3.0)
    acc = jnp.dot(act, w1_ref[...],
                  preferred_element_type=jnp.float32,
                  precision=lax.Precision.HIGHEST)
    acc += jnp.dot(cls_sel, w2_ref[...],
                   preferred_element_type=jnp.float32,
                   precision=lax.Precision.HIGHEST)
    o_ref[...] = acc + b_ref[...]


def _tc_project(act2, cls4, clsm, w1, w2, b2d):
    blk = 2048
    grid = (BATCH // blk,)
    return pl.pallas_call(
        _mm_body,
        grid=grid,
        in_specs=[
            pl.BlockSpec((blk, 128), lambda i: (i, 0)),
            pl.BlockSpec((blk, 128), lambda i: (i, 0)),
            pl.BlockSpec((blk, 1), lambda i: (i, 0)),
            pl.BlockSpec((EMBED_DIM, EMBED_DIM), lambda i: (0, 0)),
            pl.BlockSpec((CLS_DIM, EMBED_DIM), lambda i: (0, 0)),
            pl.BlockSpec((1, EMBED_DIM), lambda i: (0, 0)),
        ],
        out_specs=pl.BlockSpec((blk, EMBED_DIM), lambda i: (i, 0)),
        out_shape=jax.ShapeDtypeStruct((BATCH, EMBED_DIM), jnp.float32),
    )(act2, cls4, clsm, w1, w2, b2d)


def kernel(activity_ids, activity_classes, embedding, class_embedding, W, b):
    ids = activity_ids.astype(jnp.int32)
    cls = activity_classes.astype(jnp.int32)
    emb2 = _tc_pairize(embedding.T)
    cls2 = class_embedding.reshape(250, 128)
    act2, cls4 = _sc_gather(ids, cls // 4, emb2, cls2)
    clsm = (cls % 4).astype(jnp.float32).reshape(BATCH, 1)
    return _tc_project(act2, cls4, clsm,
                       W[:EMBED_DIM], W[EMBED_DIM:], b.reshape(1, EMBED_DIM))


# speedup vs baseline: 1.7626x; 1.2766x over previous
import functools

import jax
import jax.numpy as jnp
from jax import lax
from jax.experimental import pallas as pl
from jax.experimental.pallas import tpu as pltpu
from jax.experimental.pallas import tpu_sc as plsc

BATCH = 16384
EMBED_DIM = 64
CLS_DIM = 32
NUM_ACT = 1000000
NC = 2
NS = 16
NW = NC * NS
B_PER_W = BATCH // NW
CHUNK = 128
N_CHUNK = B_PER_W // CHUNK

TL = 4096
PAIR = 512000
TGRID = PAIR // TL


def _transpose64(x):
    return lax.dot_general(
        x.astype(jnp.bfloat16), jnp.eye(EMBED_DIM, dtype=jnp.bfloat16),
        dimension_numbers=(((0,), (0,)), ((), ())),
        preferred_element_type=jnp.float32)


def _tr_body(inl_ref, inr_ref, o_ref):
    o_ref[:, :EMBED_DIM] = _transpose64(inl_ref[...])
    o_ref[:, EMBED_DIM:] = _transpose64(inr_ref[...])


def _tc_pairize(embT):
    return pl.pallas_call(
        _tr_body,
        grid=(TGRID,),
        in_specs=[
            pl.BlockSpec((EMBED_DIM, TL), lambda i: (0, i)),
            pl.BlockSpec((EMBED_DIM, TL),
                         lambda i: (0, jnp.minimum(i + TGRID,
                                                   NUM_ACT // TL))),
        ],
        out_specs=pl.BlockSpec((TL, 128), lambda i: (i, 0)),
        out_shape=jax.ShapeDtypeStruct((PAIR, 128), jnp.float32),
    )(embT, embT)


@functools.partial(
    pl.kernel,
    out_type=(
        jax.ShapeDtypeStruct((BATCH, 128), jnp.float32),
        jax.ShapeDtypeStruct((BATCH, 128), jnp.float32),
    ),
    mesh=plsc.VectorSubcoreMesh(core_axis_name="c", subcore_axis_name="s"),
    compiler_params=pltpu.CompilerParams(use_tc_tiling_on_sc=True),
    scratch_types=[
        pltpu.VMEM((B_PER_W,), jnp.int32),
        pltpu.VMEM((B_PER_W,), jnp.int32),
        pltpu.VMEM((B_PER_W, 128), jnp.float32),
        pltpu.VMEM((B_PER_W // 2, 128), jnp.float32),
        pltpu.SemaphoreType.DMA,
        pltpu.SemaphoreType.DMA,
    ],
)
def _sc_gather(ids_hbm, cls_hbm, emb_hbm, clsemb_hbm, act_out, cls_out,
               ids_v, clsids_v, act_rows, cls_rows, sem_a, sem_c):
    wid = lax.axis_index("s") * NC + lax.axis_index("c")
    base = wid * B_PER_W
    pltpu.sync_copy(ids_hbm.at[pl.ds(base, B_PER_W)], ids_v)
    pltpu.sync_copy(cls_hbm.at[pl.ds(base, B_PER_W)], clsids_v)
    act_copies = []
    for j in range(N_CHUNK):
        act_copies.append(pltpu.async_copy(
            emb_hbm.at[ids_v.at[pl.ds(j * CHUNK, CHUNK)]],
            act_rows.at[pl.ds(j * CHUNK, CHUNK)], sem_a))
    for r in range(2):
        cls_copies = []
        for j in range(2):
            cls_copies.append(pltpu.async_copy(
                clsemb_hbm.at[clsids_v.at[pl.ds((2 * r + j) * CHUNK, CHUNK)]],
                cls_rows.at[pl.ds(j * CHUNK, CHUNK)], sem_c))
        for c in cls_copies:
            c.wait()
        pltpu.sync_copy(cls_rows,
                        cls_out.at[pl.ds(base + r * (B_PER_W // 2),
                                         B_PER_W // 2)])
    for c in act_copies:
        c.wait()
    pltpu.sync_copy(act_rows, act_out.at[pl.ds(base, B_PER_W)])


def _mm_body(act2_ref, cls4_ref, half_ref, clsm_ref, w1_ref, w2_ref, b_ref,
             o_ref):
    half = half_ref[...]
    clsm = clsm_ref[...]
    a = act2_ref[...]
    act = a[:, :EMBED_DIM] * (1.0 - half) + a[:, EMBED_DIM:] * half
    c = cls4_ref[...]
    cls_sel = c[:, 0:CLS_DIM] * (clsm == 0.0)
    cls_sel += c[:, CLS_DIM:2 * CLS_DIM] * (clsm == 1.0)
    cls_sel += c[:, 2 * CLS_DIM:3 * CLS_DIM] * (clsm == 2.0)
    cls_sel += c[:, 3 * CLS_DIM:] * (clsm == 3.0)
    acc = jnp.dot(act, w1_ref[...],
                  preferred_element_type=jnp.float32,
                  precision=lax.Precision.HIGHEST)
    acc += jnp.dot(cls_sel, w2_ref[...],
                   preferred_element_type=jnp.float32,
                   precision=lax.Precision.HIGHEST)
    o_ref[...] = acc + b_ref[...]


def _tc_project(act2, cls4, half, clsm, w1, w2, b2d):
    blk = 2048
    grid = (BATCH // blk,)
    return pl.pallas_call(
        _mm_body,
        grid=grid,
        in_specs=[
            pl.BlockSpec((blk, 128), lambda i: (i, 0)),
            pl.BlockSpec((blk, 128), lambda i: (i, 0)),
            pl.BlockSpec((blk, 1), lambda i: (i, 0)),
            pl.BlockSpec((blk, 1), lambda i: (i, 0)),
            pl.BlockSpec((EMBED_DIM, EMBED_DIM), lambda i: (0, 0)),
            pl.BlockSpec((CLS_DIM, EMBED_DIM), lambda i: (0, 0)),
            pl.BlockSpec((1, EMBED_DIM), lambda i: (0, 0)),
        ],
        out_specs=pl.BlockSpec((blk, EMBED_DIM), lambda i: (i, 0)),
        out_shape=jax.ShapeDtypeStruct((BATCH, EMBED_DIM), jnp.float32),
    )(act2, cls4, half, clsm, w1, w2, b2d)


def kernel(activity_ids, activity_classes, embedding, class_embedding, W, b):
    ids = activity_ids.astype(jnp.int32)
    cls = activity_classes.astype(jnp.int32)
    emb2 = _tc_pairize(embedding.T)
    cls2 = class_embedding.reshape(250, 128)
    act2, cls4 = _sc_gather(ids % PAIR, cls // 4, emb2, cls2)
    half = (ids >= PAIR).astype(jnp.float32).reshape(BATCH, 1)
    clsm = (cls % 4).astype(jnp.float32).reshape(BATCH, 1)
    return _tc_project(act2, cls4, half, clsm,
                       W[:EMBED_DIM], W[EMBED_DIM:], b.reshape(1, EMBED_DIM))


# speedup vs baseline: 1.8448x; 1.0467x over previous
import functools

import jax
import jax.numpy as jnp
from jax import lax
from jax.experimental import pallas as pl
from jax.experimental.pallas import tpu as pltpu
from jax.experimental.pallas import tpu_sc as plsc

BATCH = 16384
EMBED_DIM = 64
CLS_DIM = 32
NUM_ACT = 1000000
NC = 2
NS = 16
NW = NC * NS
B_PER_W = BATCH // NW
CHUNK = 128
N_CHUNK = B_PER_W // CHUNK

TL = 4096
PAIR = 512000
TGRID = PAIR // TL


def _transpose64(x):
    return lax.dot_general(
        x.astype(jnp.bfloat16), jnp.eye(EMBED_DIM, dtype=jnp.bfloat16),
        dimension_numbers=(((0,), (0,)), ((), ())),
        preferred_element_type=jnp.float32)


def _tr_body(inl_ref, inr_ref, o_ref):
    o_ref[:, :EMBED_DIM] = _transpose64(inl_ref[...])
    o_ref[:, EMBED_DIM:] = _transpose64(inr_ref[...])


def _tc_pairize(embT):
    return pl.pallas_call(
        _tr_body,
        grid=(TGRID,),
        in_specs=[
            pl.BlockSpec((EMBED_DIM, TL), lambda i: (0, i)),
            pl.BlockSpec((EMBED_DIM, TL),
                         lambda i: (0, jnp.minimum(i + TGRID,
                                                   NUM_ACT // TL))),
        ],
        out_specs=pl.BlockSpec((TL, 128), lambda i: (i, 0)),
        out_shape=jax.ShapeDtypeStruct((PAIR, 128), jnp.float32),
    )(embT, embT)


@functools.partial(
    pl.kernel,
    out_type=(
        jax.ShapeDtypeStruct((BATCH, 128), jnp.float32),
        jax.ShapeDtypeStruct((BATCH, 128), jnp.float32),
    ),
    mesh=plsc.VectorSubcoreMesh(core_axis_name="c", subcore_axis_name="s"),
    compiler_params=pltpu.CompilerParams(use_tc_tiling_on_sc=True),
    scratch_types=[
        pltpu.VMEM((B_PER_W,), jnp.int32),
        pltpu.VMEM((B_PER_W,), jnp.int32),
        pltpu.VMEM((B_PER_W, 128), jnp.float32),
        pltpu.VMEM((B_PER_W // 2, 128), jnp.float32),
        pltpu.SemaphoreType.DMA,
        pltpu.SemaphoreType.DMA,
    ],
)
def _sc_gather(ids_hbm, cls_hbm, emb_hbm, clsemb_hbm, act_out, cls_out,
               ids_v, clsids_v, act_rows, cls_rows, sem_a, sem_c):
    wid = lax.axis_index("s") * NC + lax.axis_index("c")
    base = wid * B_PER_W
    pltpu.sync_copy(ids_hbm.at[pl.ds(base, B_PER_W)], ids_v)
    pltpu.sync_copy(cls_hbm.at[pl.ds(base, B_PER_W)], clsids_v)
    act_copies = []
    for j in range(N_CHUNK):
        act_copies.append(pltpu.async_copy(
            emb_hbm.at[ids_v.at[pl.ds(j * CHUNK, CHUNK)]],
            act_rows.at[pl.ds(j * CHUNK, CHUNK)], sem_a))
    for r in range(2):
        cls_copies = []
        for j in range(2):
            cls_copies.append(pltpu.async_copy(
                clsemb_hbm.at[clsids_v.at[pl.ds((2 * r + j) * CHUNK, CHUNK)]],
                cls_rows.at[pl.ds(j * CHUNK, CHUNK)], sem_c))
        for c in cls_copies:
            c.wait()
        pltpu.sync_copy(cls_rows,
                        cls_out.at[pl.ds(base + r * (B_PER_W // 2),
                                         B_PER_W // 2)])
    for c in act_copies:
        c.wait()
    pltpu.sync_copy(act_rows, act_out.at[pl.ds(base, B_PER_W)])


def _mm_body(act2_ref, cls4_ref, sel_ref, w1_ref, w2_ref, b_ref,
             o_ref):
    sel = sel_ref[...]
    half = (sel >= 4.0).astype(jnp.float32)
    clsm = sel - 4.0 * half
    a = act2_ref[...]
    act = a[:, :EMBED_DIM] * (1.0 - half) + a[:, EMBED_DIM:] * half
    c = cls4_ref[...]
    cls_sel = c[:, 0:CLS_DIM] * (clsm == 0.0)
    cls_sel += c[:, CLS_DIM:2 * CLS_DIM] * (clsm == 1.0)
    cls_sel += c[:, 2 * CLS_DIM:3 * CLS_DIM] * (clsm == 2.0)
    cls_sel += c[:, 3 * CLS_DIM:] * (clsm == 3.0)
    acc = jnp.dot(act, w1_ref[...], preferred_element_type=jnp.float32)
    acc += jnp.dot(cls_sel, w2_ref[...], preferred_element_type=jnp.float32)
    o_ref[...] = acc + b_ref[...]


def _tc_project(act2, cls4, sel, w1, w2, b2d):
    blk = 2048
    grid = (BATCH // blk,)
    return pl.pallas_call(
        _mm_body,
        grid=grid,
        in_specs=[
            pl.BlockSpec((blk, 128), lambda i: (i, 0)),
            pl.BlockSpec((blk, 128), lambda i: (i, 0)),
            pl.BlockSpec((blk, 1), lambda i: (i, 0)),
            pl.BlockSpec((EMBED_DIM, EMBED_DIM), lambda i: (0, 0)),
            pl.BlockSpec((CLS_DIM, EMBED_DIM), lambda i: (0, 0)),
            pl.BlockSpec((1, EMBED_DIM), lambda i: (0, 0)),
        ],
        out_specs=pl.BlockSpec((blk, EMBED_DIM), lambda i: (i, 0)),
        out_shape=jax.ShapeDtypeStruct((BATCH, EMBED_DIM), jnp.float32),
    )(act2, cls4, sel, w1, w2, b2d)


def kernel(activity_ids, activity_classes, embedding, class_embedding, W, b):
    ids = activity_ids.astype(jnp.int32)
    cls = activity_classes.astype(jnp.int32)
    emb2 = _tc_pairize(embedding.T)
    cls2 = class_embedding.reshape(250, 128)
    act2, cls4 = _sc_gather(ids % PAIR, cls // 4, emb2, cls2)
    sel = (4 * (ids >= PAIR) + cls % 4).astype(jnp.float32).reshape(BATCH, 1)
    return _tc_project(act2, cls4, sel,
                       W[:EMBED_DIM], W[EMBED_DIM:], b.reshape(1, EMBED_DIM))


# speedup vs baseline: 2.0084x; 1.0887x over previous
import functools

import jax
import jax.numpy as jnp
from jax import lax
from jax.experimental import pallas as pl
from jax.experimental.pallas import tpu as pltpu
from jax.experimental.pallas import tpu_sc as plsc

BATCH = 16384
EMBED_DIM = 64
CLS_DIM = 32
NUM_ACT = 1000000
NC = 2
NS = 16
NW = NC * NS
B_PER_W = BATCH // NW
CHUNK = 128
N_CHUNK = B_PER_W // CHUNK

TL = 6400
PAIR = 512000
TGRID = PAIR // TL


def _transpose64(x):
    return lax.dot_general(
        x.astype(jnp.bfloat16), jnp.eye(EMBED_DIM, dtype=jnp.bfloat16),
        dimension_numbers=(((0,), (0,)), ((), ())),
        preferred_element_type=jnp.float32)


def _tr_body(inl_ref, inr_ref, o_ref):
    o_ref[:, :EMBED_DIM] = _transpose64(inl_ref[...])
    o_ref[:, EMBED_DIM:] = _transpose64(inr_ref[...])


def _tc_pairize(embT):
    return pl.pallas_call(
        _tr_body,
        grid=(TGRID,),
        in_specs=[
            pl.BlockSpec((EMBED_DIM, TL), lambda i: (0, i)),
            pl.BlockSpec((EMBED_DIM, TL),
                         lambda i: (0, jnp.minimum(i + TGRID,
                                                   NUM_ACT // TL))),
        ],
        out_specs=pl.BlockSpec((TL, 128), lambda i: (i, 0)),
        out_shape=jax.ShapeDtypeStruct((PAIR, 128), jnp.float32),
    )(embT, embT)


@functools.partial(
    pl.kernel,
    out_type=(
        jax.ShapeDtypeStruct((BATCH, 128), jnp.float32),
        jax.ShapeDtypeStruct((BATCH, 128), jnp.float32),
    ),
    mesh=plsc.VectorSubcoreMesh(core_axis_name="c", subcore_axis_name="s"),
    compiler_params=pltpu.CompilerParams(use_tc_tiling_on_sc=True),
    scratch_types=[
        pltpu.VMEM((B_PER_W,), jnp.int32),
        pltpu.VMEM((B_PER_W,), jnp.int32),
        pltpu.VMEM((B_PER_W, 128), jnp.float32),
        pltpu.VMEM((B_PER_W // 2, 128), jnp.float32),
        pltpu.SemaphoreType.DMA,
        pltpu.SemaphoreType.DMA,
    ],
)
def _sc_gather(ids_hbm, cls_hbm, emb_hbm, clsemb_hbm, act_out, cls_out,
               ids_v, clsids_v, act_rows, cls_rows, sem_a, sem_c):
    wid = lax.axis_index("s") * NC + lax.axis_index("c")
    base = wid * B_PER_W
    pltpu.sync_copy(ids_hbm.at[pl.ds(base, B_PER_W)], ids_v)
    pltpu.sync_copy(cls_hbm.at[pl.ds(base, B_PER_W)], clsids_v)
    act_copies = []
    for j in range(N_CHUNK):
        act_copies.append(pltpu.async_copy(
            emb_hbm.at[ids_v.at[pl.ds(j * CHUNK, CHUNK)]],
            act_rows.at[pl.ds(j * CHUNK, CHUNK)], sem_a))
    for r in range(2):
        cls_copies = []
        for j in range(2):
            cls_copies.append(pltpu.async_copy(
                clsemb_hbm.at[clsids_v.at[pl.ds((2 * r + j) * CHUNK, CHUNK)]],
                cls_rows.at[pl.ds(j * CHUNK, CHUNK)], sem_c))
        for c in cls_copies:
            c.wait()
        pltpu.sync_copy(cls_rows,
                        cls_out.at[pl.ds(base + r * (B_PER_W // 2),
                                         B_PER_W // 2)])
    for c in act_copies:
        c.wait()
    pltpu.sync_copy(act_rows, act_out.at[pl.ds(base, B_PER_W)])


def _mm_body(act2_ref, cls4_ref, sel_ref, w1_ref, w2_ref, b_ref,
             o_ref):
    sel = sel_ref[...]
    half = (sel >= 4.0).astype(jnp.float32)
    clsm = sel - 4.0 * half
    a = act2_ref[...]
    act = a[:, :EMBED_DIM] * (1.0 - half) + a[:, EMBED_DIM:] * half
    c = cls4_ref[...]
    cls_sel = c[:, 0:CLS_DIM] * (clsm == 0.0)
    cls_sel += c[:, CLS_DIM:2 * CLS_DIM] * (clsm == 1.0)
    cls_sel += c[:, 2 * CLS_DIM:3 * CLS_DIM] * (clsm == 2.0)
    cls_sel += c[:, 3 * CLS_DIM:] * (clsm == 3.0)
    acc = jnp.dot(act, w1_ref[...], preferred_element_type=jnp.float32)
    acc += jnp.dot(cls_sel, w2_ref[...], preferred_element_type=jnp.float32)
    o_ref[...] = acc + b_ref[...]


def _tc_project(act2, cls4, sel, w1, w2, b2d):
    blk = 2048
    grid = (BATCH // blk,)
    return pl.pallas_call(
        _mm_body,
        grid=grid,
        in_specs=[
            pl.BlockSpec((blk, 128), lambda i: (i, 0)),
            pl.BlockSpec((blk, 128), lambda i: (i, 0)),
            pl.BlockSpec((blk, 1), lambda i: (i, 0)),
            pl.BlockSpec((EMBED_DIM, EMBED_DIM), lambda i: (0, 0)),
            pl.BlockSpec((CLS_DIM, EMBED_DIM), lambda i: (0, 0)),
            pl.BlockSpec((1, EMBED_DIM), lambda i: (0, 0)),
        ],
        out_specs=pl.BlockSpec((blk, EMBED_DIM), lambda i: (i, 0)),
        out_shape=jax.ShapeDtypeStruct((BATCH, EMBED_DIM), jnp.float32),
    )(act2, cls4, sel, w1, w2, b2d)


def kernel(activity_ids, activity_classes, embedding, class_embedding, W, b):
    ids = activity_ids.astype(jnp.int32)
    cls = activity_classes.astype(jnp.int32)
    emb2 = _tc_pairize(embedding.T)
    cls2 = class_embedding.reshape(250, 128)
    act2, cls4 = _sc_gather(ids % PAIR, cls // 4, emb2, cls2)
    sel = (4 * (ids >= PAIR) + cls % 4).astype(jnp.float32).reshape(BATCH, 1)
    return _tc_project(act2, cls4, sel,
                       W[:EMBED_DIM], W[EMBED_DIM:], b.reshape(1, EMBED_DIM))


# speedup vs baseline: 2.2022x; 1.0965x over previous
import functools

import jax
import jax.numpy as jnp
from jax import lax
from jax.experimental import pallas as pl
from jax.experimental.pallas import tpu as pltpu
from jax.experimental.pallas import tpu_sc as plsc

BATCH = 16384
EMBED_DIM = 64
CLS_DIM = 32
NUM_ACT = 1000000
NC = 2
NS = 16
NW = NC * NS
B_PER_W = BATCH // NW
CHUNK = 128
N_CHUNK = B_PER_W // CHUNK

TL = 12800
PAIR = 512000
TGRID = PAIR // TL


def _transpose64(x):
    return lax.dot_general(
        x.astype(jnp.bfloat16), jnp.eye(EMBED_DIM, dtype=jnp.bfloat16),
        dimension_numbers=(((0,), (0,)), ((), ())),
        preferred_element_type=jnp.float32)


def _tr_body(inl_ref, inr_ref, o_ref):
    o_ref[:, :EMBED_DIM] = _transpose64(inl_ref[...])
    o_ref[:, EMBED_DIM:] = _transpose64(inr_ref[...])


def _tc_pairize(embT):
    return pl.pallas_call(
        _tr_body,
        grid=(TGRID,),
        in_specs=[
            pl.BlockSpec((EMBED_DIM, TL), lambda i: (0, i)),
            pl.BlockSpec((EMBED_DIM, TL),
                         lambda i: (0, jnp.minimum(i + TGRID,
                                                   NUM_ACT // TL))),
        ],
        out_specs=pl.BlockSpec((TL, 128), lambda i: (i, 0)),
        out_shape=jax.ShapeDtypeStruct((PAIR, 128), jnp.float32),
    )(embT, embT)


@functools.partial(
    pl.kernel,
    out_type=(
        jax.ShapeDtypeStruct((BATCH, 128), jnp.float32),
        jax.ShapeDtypeStruct((BATCH, 128), jnp.float32),
    ),
    mesh=plsc.VectorSubcoreMesh(core_axis_name="c", subcore_axis_name="s"),
    compiler_params=pltpu.CompilerParams(use_tc_tiling_on_sc=True),
    scratch_types=[
        pltpu.VMEM((B_PER_W,), jnp.int32),
        pltpu.VMEM((B_PER_W,), jnp.int32),
        pltpu.VMEM((B_PER_W, 128), jnp.float32),
        pltpu.VMEM((B_PER_W // 2, 128), jnp.float32),
        pltpu.SemaphoreType.DMA,
        pltpu.SemaphoreType.DMA,
    ],
)
def _sc_gather(ids_hbm, cls_hbm, emb_hbm, clsemb_hbm, act_out, cls_out,
               ids_v, clsids_v, act_rows, cls_rows, sem_a, sem_c):
    wid = lax.axis_index("s") * NC + lax.axis_index("c")
    base = wid * B_PER_W
    pltpu.sync_copy(ids_hbm.at[pl.ds(base, B_PER_W)], ids_v)
    pltpu.sync_copy(cls_hbm.at[pl.ds(base, B_PER_W)], clsids_v)
    act_copies = []
    for j in range(N_CHUNK):
        act_copies.append(pltpu.async_copy(
            emb_hbm.at[ids_v.at[pl.ds(j * CHUNK, CHUNK)]],
            act_rows.at[pl.ds(j * CHUNK, CHUNK)], sem_a))
    for r in range(2):
        cls_copies = []
        for j in range(2):
            cls_copies.append(pltpu.async_copy(
                clsemb_hbm.at[clsids_v.at[pl.ds((2 * r + j) * CHUNK, CHUNK)]],
                cls_rows.at[pl.ds(j * CHUNK, CHUNK)], sem_c))
        for c in cls_copies:
            c.wait()
        pltpu.sync_copy(cls_rows,
                        cls_out.at[pl.ds(base + r * (B_PER_W // 2),
                                         B_PER_W // 2)])
    for c in act_copies:
        c.wait()
    pltpu.sync_copy(act_rows, act_out.at[pl.ds(base, B_PER_W)])


def _mm_body(act2_ref, cls4_ref, sel_ref, w1_ref, w2_ref, b_ref,
             o_ref):
    sel = sel_ref[...]
    half = (sel >= 4.0).astype(jnp.float32)
    clsm = sel - 4.0 * half
    a = act2_ref[...]
    act = a[:, :EMBED_DIM] * (1.0 - half) + a[:, EMBED_DIM:] * half
    c = cls4_ref[...]
    cls_sel = c[:, 0:CLS_DIM] * (clsm == 0.0)
    cls_sel += c[:, CLS_DIM:2 * CLS_DIM] * (clsm == 1.0)
    cls_sel += c[:, 2 * CLS_DIM:3 * CLS_DIM] * (clsm == 2.0)
    cls_sel += c[:, 3 * CLS_DIM:] * (clsm == 3.0)
    acc = jnp.dot(act, w1_ref[...], preferred_element_type=jnp.float32)
    acc += jnp.dot(cls_sel, w2_ref[...], preferred_element_type=jnp.float32)
    o_ref[...] = acc + b_ref[...]


def _tc_project(act2, cls4, sel, w1, w2, b2d):
    blk = 2048
    grid = (BATCH // blk,)
    return pl.pallas_call(
        _mm_body,
        grid=grid,
        in_specs=[
            pl.BlockSpec((blk, 128), lambda i: (i, 0)),
            pl.BlockSpec((blk, 128), lambda i: (i, 0)),
            pl.BlockSpec((blk, 1), lambda i: (i, 0)),
            pl.BlockSpec((EMBED_DIM, EMBED_DIM), lambda i: (0, 0)),
            pl.BlockSpec((CLS_DIM, EMBED_DIM), lambda i: (0, 0)),
            pl.BlockSpec((1, EMBED_DIM), lambda i: (0, 0)),
        ],
        out_specs=pl.BlockSpec((blk, EMBED_DIM), lambda i: (i, 0)),
        out_shape=jax.ShapeDtypeStruct((BATCH, EMBED_DIM), jnp.float32),
    )(act2, cls4, sel, w1, w2, b2d)


def kernel(activity_ids, activity_classes, embedding, class_embedding, W, b):
    ids = activity_ids.astype(jnp.int32)
    cls = activity_classes.astype(jnp.int32)
    emb2 = _tc_pairize(embedding.T)
    cls2 = class_embedding.reshape(250, 128)
    act2, cls4 = _sc_gather(ids % PAIR, cls // 4, emb2, cls2)
    sel = (4 * (ids >= PAIR) + cls % 4).astype(jnp.float32).reshape(BATCH, 1)
    return _tc_project(act2, cls4, sel,
                       W[:EMBED_DIM], W[EMBED_DIM:], b.reshape(1, EMBED_DIM))


# speedup vs baseline: 2.2857x; 1.0379x over previous
import functools

import jax
import jax.numpy as jnp
from jax import lax
from jax.experimental import pallas as pl
from jax.experimental.pallas import tpu as pltpu
from jax.experimental.pallas import tpu_sc as plsc

BATCH = 16384
EMBED_DIM = 64
CLS_DIM = 32
NUM_ACT = 1000000
NC = 2
NS = 16
NW = NC * NS
B_PER_W = BATCH // NW
CHUNK = 128
N_CHUNK = B_PER_W // CHUNK

TL = 25600
PAIR = 512000
TGRID = PAIR // TL


def _transpose64(x):
    return lax.dot_general(
        x.astype(jnp.bfloat16), jnp.eye(EMBED_DIM, dtype=jnp.bfloat16),
        dimension_numbers=(((0,), (0,)), ((), ())),
        preferred_element_type=jnp.float32)


def _tr_body(inl_ref, inr_ref, o_ref):
    o_ref[:, :EMBED_DIM] = _transpose64(inl_ref[...])
    o_ref[:, EMBED_DIM:] = _transpose64(inr_ref[...])


def _tc_pairize(embT):
    return pl.pallas_call(
        _tr_body,
        grid=(TGRID,),
        in_specs=[
            pl.BlockSpec((EMBED_DIM, TL), lambda i: (0, i)),
            pl.BlockSpec((EMBED_DIM, TL),
                         lambda i: (0, jnp.minimum(i + TGRID,
                                                   NUM_ACT // TL))),
        ],
        out_specs=pl.BlockSpec((TL, 128), lambda i: (i, 0)),
        out_shape=jax.ShapeDtypeStruct((PAIR, 128), jnp.float32),
    )(embT, embT)


@functools.partial(
    pl.kernel,
    out_type=(
        jax.ShapeDtypeStruct((BATCH, 128), jnp.float32),
        jax.ShapeDtypeStruct((BATCH, 128), jnp.float32),
    ),
    mesh=plsc.VectorSubcoreMesh(core_axis_name="c", subcore_axis_name="s"),
    compiler_params=pltpu.CompilerParams(use_tc_tiling_on_sc=True),
    scratch_types=[
        pltpu.VMEM((B_PER_W,), jnp.int32),
        pltpu.VMEM((B_PER_W,), jnp.int32),
        pltpu.VMEM((B_PER_W, 128), jnp.float32),
        pltpu.VMEM((B_PER_W // 2, 128), jnp.float32),
        pltpu.SemaphoreType.DMA,
        pltpu.SemaphoreType.DMA,
    ],
)
def _sc_gather(ids_hbm, cls_hbm, emb_hbm, clsemb_hbm, act_out, cls_out,
               ids_v, clsids_v, act_rows, cls_rows, sem_a, sem_c):
    wid = lax.axis_index("s") * NC + lax.axis_index("c")
    base = wid * B_PER_W
    pltpu.sync_copy(ids_hbm.at[pl.ds(base, B_PER_W)], ids_v)
    pltpu.sync_copy(cls_hbm.at[pl.ds(base, B_PER_W)], clsids_v)
    act_copies = []
    for j in range(N_CHUNK):
        act_copies.append(pltpu.async_copy(
            emb_hbm.at[ids_v.at[pl.ds(j * CHUNK, CHUNK)]],
            act_rows.at[pl.ds(j * CHUNK, CHUNK)], sem_a))
    for r in range(2):
        cls_copies = []
        for j in range(2):
            cls_copies.append(pltpu.async_copy(
                clsemb_hbm.at[clsids_v.at[pl.ds((2 * r + j) * CHUNK, CHUNK)]],
                cls_rows.at[pl.ds(j * CHUNK, CHUNK)], sem_c))
        for c in cls_copies:
            c.wait()
        pltpu.sync_copy(cls_rows,
                        cls_out.at[pl.ds(base + r * (B_PER_W // 2),
                                         B_PER_W // 2)])
    for c in act_copies:
        c.wait()
    pltpu.sync_copy(act_rows, act_out.at[pl.ds(base, B_PER_W)])


def _mm_body(act2_ref, cls4_ref, sel_ref, w1_ref, w2_ref, b_ref,
             o_ref):
    sel = sel_ref[...]
    half = (sel >= 4.0).astype(jnp.float32)
    clsm = sel - 4.0 * half
    a = act2_ref[...]
    act = a[:, :EMBED_DIM] * (1.0 - half) + a[:, EMBED_DIM:] * half
    c = cls4_ref[...]
    cls_sel = c[:, 0:CLS_DIM] * (clsm == 0.0)
    cls_sel += c[:, CLS_DIM:2 * CLS_DIM] * (clsm == 1.0)
    cls_sel += c[:, 2 * CLS_DIM:3 * CLS_DIM] * (clsm == 2.0)
    cls_sel += c[:, 3 * CLS_DIM:] * (clsm == 3.0)
    acc = jnp.dot(act, w1_ref[...], preferred_element_type=jnp.float32)
    acc += jnp.dot(cls_sel, w2_ref[...], preferred_element_type=jnp.float32)
    o_ref[...] = acc + b_ref[...]


def _tc_project(act2, cls4, sel, w1, w2, b2d):
    blk = 2048
    grid = (BATCH // blk,)
    return pl.pallas_call(
        _mm_body,
        grid=grid,
        in_specs=[
            pl.BlockSpec((blk, 128), lambda i: (i, 0)),
            pl.BlockSpec((blk, 128), lambda i: (i, 0)),
            pl.BlockSpec((blk, 1), lambda i: (i, 0)),
            pl.BlockSpec((EMBED_DIM, EMBED_DIM), lambda i: (0, 0)),
            pl.BlockSpec((CLS_DIM, EMBED_DIM), lambda i: (0, 0)),
            pl.BlockSpec((1, EMBED_DIM), lambda i: (0, 0)),
        ],
        out_specs=pl.BlockSpec((blk, EMBED_DIM), lambda i: (i, 0)),
        out_shape=jax.ShapeDtypeStruct((BATCH, EMBED_DIM), jnp.float32),
    )(act2, cls4, sel, w1, w2, b2d)


def kernel(activity_ids, activity_classes, embedding, class_embedding, W, b):
    ids = activity_ids.astype(jnp.int32)
    cls = activity_classes.astype(jnp.int32)
    emb2 = _tc_pairize(embedding.T)
    cls2 = class_embedding.reshape(250, 128)
    act2, cls4 = _sc_gather(ids % PAIR, cls // 4, emb2, cls2)
    sel = (4 * (ids >= PAIR) + cls % 4).astype(jnp.float32).reshape(BATCH, 1)
    return _tc_project(act2, cls4, sel,
                       W[:EMBED_DIM], W[EMBED_DIM:], b.reshape(1, EMBED_DIM))
